# baseline (speedup 1.0000x reference)
import functools

import jax
import jax.numpy as jnp
from jax import lax
from jax.experimental import pallas as pl
from jax.experimental.pallas import tpu as pltpu
from jax.experimental.pallas import tpu_sc as plsc

EPS = 1e-12
UNROLL = 8


def _rsqrt(x):
    bits = lax.bitcast_convert_type(x, jnp.int32)
    y = lax.bitcast_convert_type(
        jnp.int32(0x5F3759DF) - lax.shift_right_arithmetic(bits, 1), jnp.float32
    )
    for _ in range(3):
        y = y * (1.5 - 0.5 * x * y * y)
    return y


def _build(B, L, V, D):
    NC, NS = 2, 16
    NW = NC * NS
    seqs_per_w = B // NW
    mesh = plsc.VectorSubcoreMesh(core_axis_name="c", subcore_axis_name="s")

    @functools.partial(
        pl.kernel,
        out_type=jax.ShapeDtypeStruct((B * L, D), jnp.float32),
        mesh=mesh,
        compiler_params=pltpu.CompilerParams(use_tc_tiling_on_sc=False),
        scratch_types=[
            pltpu.VMEM((L,), jnp.int32),
            pltpu.VMEM((L, D), jnp.float32),
            pltpu.VMEM((L, D), jnp.float32),
            pltpu.VMEM((2, D), jnp.float32),
            pltpu.VMEM((D,), jnp.float32),
            pltpu.VMEM((D,), jnp.float32),
            pltpu.SemaphoreType.DMA,
        ],
    )
    def k(ids_hbm, word_hbm, pos_hbm, type_hbm, gamma_hbm, beta_hbm, out_hbm,
          idx_v, rows_v, comb_v, type_v, gamma_v, beta_v, sem):
        wid = lax.axis_index("s") * NC + lax.axis_index("c")

        pltpu.sync_copy(pos_hbm, comb_v)
        pltpu.sync_copy(type_hbm, type_v)
        pltpu.sync_copy(gamma_hbm, gamma_v)
        pltpu.sync_copy(beta_hbm, beta_v)

        nj = D // 16
        tvec = [type_v[0, pl.ds(j * 16, 16)] for j in range(nj)]

        def pre(l, _):
            for j in range(nj):
                comb_v[l, pl.ds(j * 16, 16)] = comb_v[l, pl.ds(j * 16, 16)] + tvec[j]
            return 0

        lax.fori_loop(0, L, pre, 0)

        gvec = [gamma_v[pl.ds(j * 16, 16)] for j in range(nj)]
        bvec = [beta_v[pl.ds(j * 16, 16)] for j in range(nj)]

        lane = lax.iota(jnp.int32, 16)
        perms = [lane ^ sh for sh in (8, 4, 2, 1)]

        def allsum(v):
            for p in perms:
                v = v + v.at[p].get(mode="promise_in_bounds", unique_indices=True)
            return v

        def one_token(t):
            x = [rows_v[t, pl.ds(j * 16, 16)] + comb_v[t, pl.ds(j * 16, 16)]
                 for j in range(nj)]
            s = allsum((x[0] + x[1]) + (x[2] + x[3]))
            q = allsum((x[0] * x[0] + x[1] * x[1])
                       + (x[2] * x[2] + x[3] * x[3]))
            mean = s * (1.0 / D)
            var = q * (1.0 / D) - mean * mean
            rstd = _rsqrt(var + EPS)
            mr = mean * rstd
            for j in range(nj):
                rows_v[t, pl.ds(j * 16, 16)] = (x[j] * rstd - mr) * gvec[j] + bvec[j]

        def seq_body(si, _):
            b = wid * seqs_per_w + si
            pltpu.sync_copy(ids_hbm.at[b], idx_v)
            copies = [
                pltpu.async_copy(
                    word_hbm.at[idx_v.at[pl.ds(c * 128, 128)]],
                    rows_v.at[pl.ds(c * 128, 128)],
                    sem,
                )
                for c in range(L // 128)
            ]
            for cp in copies:
                cp.wait()

            def tok_group(g, _):
                for u in range(UNROLL):
                    one_token(g * UNROLL + u)
                return 0

            lax.fori_loop(0, L // UNROLL, tok_group, 0)
            pltpu.sync_copy(rows_v, out_hbm.at[pl.ds(b * L, L)])
            return 0

        lax.fori_loop(0, seqs_per_w, seq_body, 0)

    return k


def kernel(input_ids, word_emb, pos_emb, type_emb, ln_gamma, ln_beta):
    B, L = input_ids.shape
    V, D = word_emb.shape
    k = _build(B, L, V, D)
    out = k(input_ids, word_emb, pos_emb, type_emb, ln_gamma, ln_beta)
    return out.reshape(B, L, D)
